# baseline (speedup 1.0000x reference)
import functools

import jax
import jax.numpy as jnp
from jax import lax
from jax.experimental import pallas as pl
from jax.experimental.pallas import tpu as pltpu
from jax.experimental.pallas import tpu_sc as plsc

HIDDEN = 64
EPS = 1e-5
L = 16
NC, NS = 2, 16
NW = NC * NS
GCHUNK = 128
GROUP = 512
GPG = GROUP // GCHUNK


def _rsqrt(x):
    i = lax.bitcast_convert_type(x, jnp.int32)
    i = jnp.int32(0x5F3759DF) - (i >> 1)
    y = lax.bitcast_convert_type(i, jnp.float32)
    for _ in range(3):
        y = y * (1.5 - 0.5 * x * y * y)
    return y


def _make_sc_kernel(B):
    per_tile = B // NW
    ngroups = per_tile // GROUP
    nchunks = per_tile // GCHUNK
    mesh = plsc.VectorSubcoreMesh(
        core_axis_name="c", subcore_axis_name="s",
        num_cores=NC, num_subcores=NS)

    @functools.partial(
        pl.kernel,
        out_type=jax.ShapeDtypeStruct((B, HIDDEN), jnp.float32),
        mesh=mesh,
        scratch_types=[
            pltpu.VMEM((nchunks, GCHUNK), jnp.int32),
            pltpu.VMEM((GROUP, HIDDEN), jnp.float32),
            pltpu.VMEM((HIDDEN,), jnp.float32),
            pltpu.VMEM((HIDDEN,), jnp.float32),
            pltpu.SemaphoreType.DMA,
        ],
        compiler_params=pltpu.CompilerParams(
            needs_layout_passes=False, use_tc_tiling_on_sc=False),
    )
    def sc_kernel(idx_hbm, table_hbm, gamma_hbm, beta_hbm, out_hbm,
                  idx_v, rows_v, gamma_v, beta_v, gsem):
        wid = lax.axis_index("s") * NC + lax.axis_index("c")
        base = wid * per_tile

        pltpu.sync_copy(idx_hbm.at[pl.ds(wid * nchunks, nchunks)], idx_v)
        pltpu.sync_copy(gamma_hbm, gamma_v)
        pltpu.sync_copy(beta_hbm, beta_v)

        g4 = [gamma_v[pl.ds(q * L, L)] for q in range(4)]
        b4 = [beta_v[pl.ds(q * L, L)] for q in range(4)]

        def row_body(r, _):
            x = [rows_v[r, pl.ds(q * L, L)] for q in range(4)]
            total = jnp.sum(x[0] + x[1] + x[2] + x[3])
            sq = jnp.sum(x[0] * x[0] + x[1] * x[1]
                         + x[2] * x[2] + x[3] * x[3])
            mean = total * (1.0 / HIDDEN)
            var = sq * (1.0 / HIDDEN) - mean * mean
            rstd = _rsqrt(var + EPS)
            for q in range(4):
                rows_v[r, pl.ds(q * L, L)] = (x[q] - mean) * rstd * g4[q] + b4[q]
            return 0

        def group_body(g, _):
            for j in range(GPG):
                pltpu.async_copy(
                    table_hbm.at[idx_v.at[g * GPG + j]],
                    rows_v.at[pl.ds(j * GCHUNK, GCHUNK)],
                    gsem).wait()
            lax.fori_loop(0, GROUP, row_body, 0, unroll=4)
            pltpu.sync_copy(
                rows_v,
                out_hbm.at[pl.ds(base + g * GROUP, GROUP)])
            return 0

        lax.fori_loop(0, ngroups, group_body, 0)

    return sc_kernel


def kernel(input_idx, table, ln_gamma, ln_beta):
    nb, nt = input_idx.shape
    B = nb * nt
    idx2d = input_idx.reshape(B // GCHUNK, GCHUNK).astype(jnp.int32)
    out = _make_sc_kernel(B)(idx2d, table, ln_gamma, ln_beta)
    return out.reshape(nb, nt, HIDDEN)
